# baseline (speedup 1.0000x reference)
import functools

import jax
import jax.numpy as jnp
from jax.experimental import pallas as pl
from jax.experimental.pallas import tpu as pltpu

_NC = 8


def _lstm_kernel(xs_ref, wc_ref, wfc_ref, out_ref, *scratch,
                 Tc, NT, H, HB, I):
    inps = scratch[0:_NC]
    cs = scratch[_NC:2 * _NC]
    j = pl.program_id(1)

    @pl.when(j == 0)
    def _prologue():
        for k in range(_NC):
            inps[k][I + 1:8, :] = jnp.zeros((8 - I - 1, HB), jnp.bfloat16)
            inps[k][8:8 + H, :] = jnp.zeros((H, HB), jnp.bfloat16)
            cs[k][...] = jnp.zeros((H, HB), jnp.float32)

    def step(t, _):
        for k in range(_NC):
            inp_ref, c_ref = inps[k], cs[k]
            inp_ref[0:I + 1, :] = xs_ref[t, :, k * HB:(k + 1) * HB]
            g = jnp.dot(wc_ref[...], inp_ref[...],
                        preferred_element_type=jnp.float32)
            ti = jnp.tanh(g[0:H])
            tf = jnp.tanh(g[H:2 * H])
            gg = jnp.tanh(g[2 * H:3 * H])
            to = jnp.tanh(g[3 * H:4 * H])
            c_old = c_ref[...]
            c = 0.5 * ((tf * c_old + c_old) + (ti * gg + gg))
            c_ref[...] = c
            tc = jnp.tanh(c)
            inp_ref[8:8 + H, :] = ((to + 1.0) * tc).astype(jnp.bfloat16)
        return ()

    jax.lax.fori_loop(0, Tc, step, (), unroll=64)

    @pl.when(j == NT - 1)
    def _epilogue():
        for k in range(_NC):
            out_ref[:, k * HB:(k + 1) * HB] = jnp.dot(
                wfc_ref[...], inps[k][...],
                preferred_element_type=jnp.float32)


def kernel(x, W_ih, W_hh, b_ih, b_hh, W_fc, b_fc):
    B, T, I = x.shape
    H = W_hh.shape[1]
    O = W_fc.shape[0]
    K = 8 + H

    xT = jnp.transpose(x.astype(jnp.bfloat16), (1, 2, 0))
    xTp = jnp.concatenate([xT, jnp.ones((T, 1, B), jnp.bfloat16)], axis=1)

    Wc = jnp.zeros((4 * H, K), jnp.float32)
    Wc = Wc.at[:, 0:I].set(W_ih)
    Wc = Wc.at[:, I].set(b_ih + b_hh)
    Wc = Wc.at[:, 8:K].set(W_hh * 0.5)
    gate_scale = jnp.concatenate(
        [jnp.full((2 * H, 1), 0.5, jnp.float32),
         jnp.ones((H, 1), jnp.float32),
         jnp.full((H, 1), 0.5, jnp.float32)], axis=0)
    Wc = (Wc * gate_scale).astype(jnp.bfloat16)

    Wfcp = jnp.zeros((8, K), jnp.float32)
    Wfcp = Wfcp.at[0:O, 8:K].set(W_fc * 0.5)
    Wfcp = Wfcp.at[0:O, I].set(b_fc)
    Wfcp = Wfcp.astype(jnp.bfloat16)

    block_B = min(2048, B)
    HB = block_B // _NC
    Tc = 64
    NT = T // Tc
    grid = (B // block_B, NT)

    out = pl.pallas_call(
        functools.partial(_lstm_kernel, Tc=Tc, NT=NT, H=H, HB=HB, I=I),
        out_shape=jax.ShapeDtypeStruct((8, B), jnp.float32),
        grid=grid,
        in_specs=[
            pl.BlockSpec((Tc, I + 1, block_B), lambda i, j: (j, 0, i)),
            pl.BlockSpec((4 * H, K), lambda i, j: (0, 0)),
            pl.BlockSpec((8, K), lambda i, j: (0, 0)),
        ],
        out_specs=pl.BlockSpec((8, block_B), lambda i, j: (0, i)),
        scratch_shapes=(
            [pltpu.VMEM((K, HB), jnp.bfloat16) for _ in range(_NC)]
            + [pltpu.VMEM((H, HB), jnp.float32) for _ in range(_NC)]
        ),
        compiler_params=pltpu.CompilerParams(
            dimension_semantics=("parallel", "arbitrary"),
            vmem_limit_bytes=48 * 1024 * 1024,
        ),
        name="fatigue_lstm",
    )(xTp, Wc, Wfcp)

    return out[:O].T
